# baseline (speedup 1.0000x reference)
import functools
import math

import jax
import jax.numpy as jnp
from jax import lax
from jax.experimental import pallas as pl
from jax.experimental.pallas import tpu as pltpu
from jax.experimental.pallas import tpu_sc as plsc

_LN2 = math.log(2.0)


def _ssp(x):
    return jnp.maximum(x, 0.0) + jnp.log(1.0 + jnp.exp(-jnp.abs(x))) - _LN2


def _dot(a, b):
    return jnp.dot(a, b, preferred_element_type=jnp.float32)


def _g_all(desc, wgs):
    E, K = desc.shape
    F = wgs[0].shape[1]
    nbl = len(wgs)
    EB = 2000
    grid = E // EB

    def body(desc_ref, *refs):
        w_refs = refs[:nbl]
        out_refs = refs[nbl:]
        d = desc_ref[...]
        for b in range(nbl):
            out_refs[b][...] = _dot(d, w_refs[b][...])

    return pl.pallas_call(
        body,
        grid=(grid,),
        in_specs=[pl.BlockSpec((EB, K), lambda i: (i, 0))]
        + [pl.BlockSpec((K, F), lambda i: (0, 0))] * nbl,
        out_specs=[pl.BlockSpec((EB, F), lambda i: (i, 0))] * nbl,
        out_shape=[jax.ShapeDtypeStruct((E, F), jnp.float32)] * nbl,
    )(desc, *wgs)


def _node_pre(x, wi, bi, wj, bj):
    N, F = x.shape
    RB = 2000
    grid = N // RB

    def body(x_ref, wi_ref, bi_ref, wj_ref, bj_ref, xi_ref, xj_ref):
        xa = _ssp(x_ref[...])
        xi_ref[...] = _ssp(_dot(xa, wi_ref[...]) + bi_ref[...])
        xj_ref[...] = _ssp(_dot(xa, wj_ref[...]) + bj_ref[...])

    wspec = pl.BlockSpec((F, F), lambda i: (0, 0))
    bspec = pl.BlockSpec((1, F), lambda i: (0, 0))
    rspec = pl.BlockSpec((RB, F), lambda i: (i, 0))
    return pl.pallas_call(
        body,
        grid=(grid,),
        in_specs=[rspec, wspec, bspec, wspec, bspec],
        out_specs=[rspec, rspec],
        out_shape=[jax.ShapeDtypeStruct((N, F), jnp.float32)] * 2,
    )(x, wi, bi.reshape(1, F), wj, bj.reshape(1, F))


def _node_post(x, xi, agg2, p, p_next=None):
    N, F = x.shape
    RB = 2000
    grid = N // RB
    nri = len(p['res_int'])
    nra = len(p['res_at'])

    res_flat = []
    for (w1, b1, w2, b2) in p['res_int'] + p['res_at']:
        res_flat += [w1, b1.reshape(1, F), w2, b2.reshape(1, F)]

    extra = []
    n_out = 1
    if p_next is not None:
        extra = [p_next['Wi'], p_next['bi'].reshape(1, F),
                 p_next['Wj'], p_next['bj'].reshape(1, F)]
        n_out = 3

    def body(*refs):
        x_ref, xi_ref, a0_ref, a1_ref, u_ref, wo_ref, bo_ref = refs[:7]
        rrefs = refs[7:7 + 4 * (nri + nra)]
        out_refs = refs[len(refs) - n_out:]

        def residual(v, k):
            w1, b1, w2, b2 = rrefs[4 * k:4 * k + 4]
            y = _ssp(v)
            y = _ssp(_dot(y, w1[...]) + b1[...])
            y = _dot(y, w2[...]) + b2[...]
            return v + y

        m = xi_ref[...] + a0_ref[...] + a1_ref[...]
        for t in range(nri):
            m = residual(m, t)
        m = _ssp(m)
        xn = u_ref[...] * x_ref[...] + _dot(m, wo_ref[...]) + bo_ref[...]
        for t in range(nra):
            xn = residual(xn, nri + t)
        out_refs[0][...] = xn
        if p_next is not None:
            wi_ref, bi_ref, wj_ref, bj_ref = refs[7 + 4 * (nri + nra):
                                                  11 + 4 * (nri + nra)]
            xa = _ssp(xn)
            out_refs[1][...] = _ssp(_dot(xa, wi_ref[...]) + bi_ref[...])
            out_refs[2][...] = _ssp(_dot(xa, wj_ref[...]) + bj_ref[...])

    rspec = pl.BlockSpec((RB, F), lambda i: (i, 0))
    wspec = pl.BlockSpec((F, F), lambda i: (0, 0))
    bspec = pl.BlockSpec((1, F), lambda i: (0, 0))
    nblk = N // RB
    a0spec = pl.BlockSpec((RB, F), lambda i: (i, 0))
    a1spec = pl.BlockSpec((RB, F), lambda i: (i + nblk, 0))
    res_specs = []
    for _ in range(nri + nra):
        res_specs += [wspec, bspec, wspec, bspec]
    extra_specs = [wspec, bspec, wspec, bspec] if p_next is not None else []
    out = pl.pallas_call(
        body,
        grid=(grid,),
        in_specs=([rspec, rspec, a0spec, a1spec, bspec, wspec, bspec]
                  + res_specs + extra_specs),
        out_specs=[rspec] * n_out,
        out_shape=[jax.ShapeDtypeStruct((N, F), jnp.float32)] * n_out,
    )(x, xi, agg2, agg2, p['u'].reshape(1, F), p['Wo'], p['bo'].reshape(1, F),
      *res_flat, *extra)
    return out if p_next is not None else out[0]


def _edge_sc(g, xj, idx_i, idx_j, zeros_nf):
    E, F = g.shape
    N = xj.shape[0]
    NC, NS = 2, 16
    NW = NC * NS
    EPW = E // NW
    C = 80
    NCH = EPW // C
    assert NCH >= 5 and NCH % 2 == 1
    RPS = (N // NS) // 8 * 8
    RPS_LAST = N - (NS - 1) * RPS

    mesh = plsc.VectorSubcoreMesh(core_axis_name="c", subcore_axis_name="s")

    @functools.partial(
        pl.kernel,
        out_type=jax.ShapeDtypeStruct((NC * N, F), jnp.float32),
        mesh=mesh,
        scratch_types=[
            pltpu.VMEM((2, C), jnp.int32),
            pltpu.VMEM((2, C), jnp.int32),
            pltpu.VMEM((C, F), jnp.float32),
            pltpu.VMEM((C, F), jnp.float32),
            pltpu.VMEM((C, F), jnp.float32),
            pltpu.VMEM((C, F), jnp.float32),
            pltpu.VMEM_SHARED((N, F), jnp.float32),
            pltpu.SemaphoreType.DMA,
            pltpu.SemaphoreType.DMA,
        ],
    )
    def ek(g_hbm, xj_hbm, ii_hbm, ij_hbm, z_hbm, out_hbm,
           ij2, ii2, g0, g1, r0, r1, agg_sh,
           isem, dsem):
        cid = lax.axis_index("c")
        sid = lax.axis_index("s")
        wid = sid * NC + cid

        @pl.when(sid < NS - 1)
        def _z_body():
            pltpu.sync_copy(z_hbm.at[pl.ds(sid * RPS, RPS)],
                            agg_sh.at[pl.ds(sid * RPS, RPS)])

        @pl.when(sid == NS - 1)
        def _z_last():
            pltpu.sync_copy(z_hbm.at[pl.ds((NS - 1) * RPS, RPS_LAST)],
                            agg_sh.at[pl.ds((NS - 1) * RPS, RPS_LAST)])

        plsc.subcore_barrier()

        base = wid * EPW
        gb = (g0, g1)
        rb = (r0, r1)

        def idx_issue(k, p):
            pltpu.make_async_copy(
                ij_hbm.at[pl.ds(base + k * C, C)], ij2.at[p], isem).start()
            pltpu.make_async_copy(
                ii_hbm.at[pl.ds(base + k * C, C)], ii2.at[p], isem).start()

        def idx_wait(k, p):
            pltpu.make_async_copy(
                ij_hbm.at[pl.ds(base + k * C, C)], ij2.at[p], isem).wait()
            pltpu.make_async_copy(
                ii_hbm.at[pl.ds(base + k * C, C)], ii2.at[p], isem).wait()

        def in_issue(k, p):
            pltpu.make_async_copy(
                xj_hbm.at[ij2.at[p]], rb[p], dsem).start()
            pltpu.make_async_copy(
                g_hbm.at[pl.ds(base + k * C, C)], gb[p], dsem).start()

        def in_wait(k, p):
            pltpu.make_async_copy(
                xj_hbm.at[ij2.at[p]], rb[p], dsem).wait()
            pltpu.make_async_copy(
                g_hbm.at[pl.ds(base + k * C, C)], gb[p], dsem).wait()

        def mul_scatter(p):
            @plsc.parallel_loop(0, C, step=8)
            def _rows(r):
                for dr in range(8):
                    for t in range(F // 16):
                        s0, s1 = r + dr, pl.ds(16 * t, 16)
                        gb[p].at[s0, s1][...] = (
                            gb[p].at[s0, s1][...] * rb[p].at[s0, s1][...])

            pltpu.sync_copy(gb[p], agg_sh.at[ii2.at[p]], add=True)

        def chunk_step(k, p, issue_next):
            if issue_next:
                idx_issue(k + 1, 1 - p)
            in_wait(k, p)
            if issue_next:
                idx_wait(k + 1, 1 - p)
                in_issue(k + 1, 1 - p)
            mul_scatter(p)

        idx_issue(0, 0)
        idx_wait(0, 0)
        in_issue(0, 0)

        @pl.loop(0, NCH - 1, step=2)
        def _pair(k):
            chunk_step(k, 0, True)
            chunk_step(k + 1, 1, True)

        chunk_step(NCH - 1, 0, False)

        plsc.subcore_barrier()

        @pl.when(sid < NS - 1)
        def _f_body():
            pltpu.sync_copy(agg_sh.at[pl.ds(sid * RPS, RPS)],
                            out_hbm.at[pl.ds(cid * N + sid * RPS, RPS)])

        @pl.when(sid == NS - 1)
        def _f_last():
            pltpu.sync_copy(agg_sh.at[pl.ds((NS - 1) * RPS, RPS_LAST)],
                            out_hbm.at[pl.ds(cid * N + (NS - 1) * RPS, RPS_LAST)])

    return ek(g, xj, idx_i, idx_j, zeros_nf)


def kernel(features, descriptors, idx_i, idx_j, params):
    N, F = features.shape
    idx_i = idx_i.astype(jnp.int32)
    idx_j = idx_j.astype(jnp.int32)
    zeros_nf = jnp.zeros((N, F), jnp.float32)

    (g_first,) = _g_all(descriptors, [params[0]['Wg']])
    g_rest = _g_all(descriptors, [p['Wg'] for p in params[1:]])
    g_list = [g_first] + list(g_rest)

    x = features
    outs = []
    xi, xj = _node_pre(x, params[0]['Wi'], params[0]['bi'],
                       params[0]['Wj'], params[0]['bj'])
    for b, p in enumerate(params):
        agg2 = _edge_sc(g_list[b], xj, idx_i, idx_j, zeros_nf)
        if b + 1 < len(params):
            x, xi, xj = _node_post(x, xi, agg2, p, params[b + 1])
        else:
            x = _node_post(x, xi, agg2, p)
        outs.append(x)
    return tuple(outs)


# speedup vs baseline: 1.1259x; 1.1259x over previous
import functools
import math

import jax
import jax.numpy as jnp
from jax import lax
from jax.experimental import pallas as pl
from jax.experimental.pallas import tpu as pltpu
from jax.experimental.pallas import tpu_sc as plsc

_LN2 = math.log(2.0)


def _ssp(x):
    return jnp.maximum(x, 0.0) + jnp.log(1.0 + jnp.exp(-jnp.abs(x))) - _LN2


def _dot(a, b):
    return jnp.dot(a, b, preferred_element_type=jnp.float32)


def _g_all(desc, wgs):
    E, K = desc.shape
    F = wgs[0].shape[1]
    nbl = len(wgs)
    EB = 2000
    grid = E // EB

    def body(desc_ref, *refs):
        w_refs = refs[:nbl]
        out_refs = refs[nbl:]
        d = desc_ref[...]
        for b in range(nbl):
            out_refs[b][...] = _dot(d, w_refs[b][...])

    return pl.pallas_call(
        body,
        grid=(grid,),
        in_specs=[pl.BlockSpec((EB, K), lambda i: (i, 0))]
        + [pl.BlockSpec((K, F), lambda i: (0, 0))] * nbl,
        out_specs=[pl.BlockSpec((EB, F), lambda i: (i, 0))] * nbl,
        out_shape=[jax.ShapeDtypeStruct((E, F), jnp.float32)] * nbl,
    )(desc, *wgs)


def _node_pre(x, wi, bi, wj, bj):
    N, F = x.shape
    RB = 2000
    grid = N // RB

    def body(x_ref, wi_ref, bi_ref, wj_ref, bj_ref, xi_ref, xj_ref):
        xa = _ssp(x_ref[...])
        xi_ref[...] = _ssp(_dot(xa, wi_ref[...]) + bi_ref[...])
        xj_ref[...] = _ssp(_dot(xa, wj_ref[...]) + bj_ref[...])

    wspec = pl.BlockSpec((F, F), lambda i: (0, 0))
    bspec = pl.BlockSpec((1, F), lambda i: (0, 0))
    rspec = pl.BlockSpec((RB, F), lambda i: (i, 0))
    return pl.pallas_call(
        body,
        grid=(grid,),
        in_specs=[rspec, wspec, bspec, wspec, bspec],
        out_specs=[rspec, rspec],
        out_shape=[jax.ShapeDtypeStruct((N, F), jnp.float32)] * 2,
    )(x, wi, bi.reshape(1, F), wj, bj.reshape(1, F))


def _node_post(x, xi, agg2, p, p_next=None):
    N, F = x.shape
    RB = 2000
    grid = N // RB
    nri = len(p['res_int'])
    nra = len(p['res_at'])

    res_flat = []
    for (w1, b1, w2, b2) in p['res_int'] + p['res_at']:
        res_flat += [w1, b1.reshape(1, F), w2, b2.reshape(1, F)]

    extra = []
    n_out = 1
    if p_next is not None:
        extra = [p_next['Wi'], p_next['bi'].reshape(1, F),
                 p_next['Wj'], p_next['bj'].reshape(1, F)]
        n_out = 3

    def body(*refs):
        x_ref, xi_ref, a0_ref, a1_ref, u_ref, wo_ref, bo_ref = refs[:7]
        rrefs = refs[7:7 + 4 * (nri + nra)]
        out_refs = refs[len(refs) - n_out:]

        def residual(v, k):
            w1, b1, w2, b2 = rrefs[4 * k:4 * k + 4]
            y = _ssp(v)
            y = _ssp(_dot(y, w1[...]) + b1[...])
            y = _dot(y, w2[...]) + b2[...]
            return v + y

        m = xi_ref[...] + a0_ref[...] + a1_ref[...]
        for t in range(nri):
            m = residual(m, t)
        m = _ssp(m)
        xn = u_ref[...] * x_ref[...] + _dot(m, wo_ref[...]) + bo_ref[...]
        for t in range(nra):
            xn = residual(xn, nri + t)
        out_refs[0][...] = xn
        if p_next is not None:
            wi_ref, bi_ref, wj_ref, bj_ref = refs[7 + 4 * (nri + nra):
                                                  11 + 4 * (nri + nra)]
            xa = _ssp(xn)
            out_refs[1][...] = _ssp(_dot(xa, wi_ref[...]) + bi_ref[...])
            out_refs[2][...] = _ssp(_dot(xa, wj_ref[...]) + bj_ref[...])

    rspec = pl.BlockSpec((RB, F), lambda i: (i, 0))
    wspec = pl.BlockSpec((F, F), lambda i: (0, 0))
    bspec = pl.BlockSpec((1, F), lambda i: (0, 0))
    nblk = N // RB
    a0spec = pl.BlockSpec((RB, F), lambda i: (i, 0))
    a1spec = pl.BlockSpec((RB, F), lambda i: (i + nblk, 0))
    res_specs = []
    for _ in range(nri + nra):
        res_specs += [wspec, bspec, wspec, bspec]
    extra_specs = [wspec, bspec, wspec, bspec] if p_next is not None else []
    out = pl.pallas_call(
        body,
        grid=(grid,),
        in_specs=([rspec, rspec, a0spec, a1spec, bspec, wspec, bspec]
                  + res_specs + extra_specs),
        out_specs=[rspec] * n_out,
        out_shape=[jax.ShapeDtypeStruct((N, F), jnp.float32)] * n_out,
    )(x, xi, agg2, agg2, p['u'].reshape(1, F), p['Wo'], p['bo'].reshape(1, F),
      *res_flat, *extra)
    return out if p_next is not None else out[0]


def _edge_sc(g, xj, idx_i, idx_j, zeros_nf):
    E, F = g.shape
    N = xj.shape[0]
    NC, NS = 2, 16
    NW = NC * NS
    EPW = E // NW
    C = 80
    NCH = EPW // C
    assert NCH >= 5 and NCH % 2 == 1
    RPS = (N // NS) // 8 * 8
    RPS_LAST = N - (NS - 1) * RPS

    mesh = plsc.VectorSubcoreMesh(core_axis_name="c", subcore_axis_name="s")

    @functools.partial(
        pl.kernel,
        out_type=jax.ShapeDtypeStruct((NC * N, F), jnp.float32),
        mesh=mesh,
        scratch_types=[
            pltpu.VMEM((2, C), jnp.int32),
            pltpu.VMEM((2, C), jnp.int32),
            pltpu.VMEM((C, F), jnp.float32),
            pltpu.VMEM((C, F), jnp.float32),
            pltpu.VMEM((C, F), jnp.float32),
            pltpu.VMEM((C, F), jnp.float32),
            pltpu.VMEM_SHARED((N, F), jnp.float32),
            pltpu.SemaphoreType.DMA,
            pltpu.SemaphoreType.DMA,
        ],
    )
    def ek(g_hbm, xj_hbm, ii_hbm, ij_hbm, z_hbm, out_hbm,
           ij2, ii2, g0, g1, r0, r1, agg_sh,
           isem, dsem):
        cid = lax.axis_index("c")
        sid = lax.axis_index("s")
        wid = sid * NC + cid

        @pl.when(sid < NS - 1)
        def _z_body():
            pltpu.sync_copy(z_hbm.at[pl.ds(sid * RPS, RPS)],
                            agg_sh.at[pl.ds(sid * RPS, RPS)])

        @pl.when(sid == NS - 1)
        def _z_last():
            pltpu.sync_copy(z_hbm.at[pl.ds((NS - 1) * RPS, RPS_LAST)],
                            agg_sh.at[pl.ds((NS - 1) * RPS, RPS_LAST)])

        plsc.subcore_barrier()

        base = wid * EPW
        gb = (g0, g1)
        rb = (r0, r1)

        def idx_issue(k, p):
            pltpu.make_async_copy(
                ij_hbm.at[pl.ds(base + k * C, C)], ij2.at[p], isem).start()
            pltpu.make_async_copy(
                ii_hbm.at[pl.ds(base + k * C, C)], ii2.at[p], isem).start()

        def idx_wait(k, p):
            pltpu.make_async_copy(
                ij_hbm.at[pl.ds(base + k * C, C)], ij2.at[p], isem).wait()
            pltpu.make_async_copy(
                ii_hbm.at[pl.ds(base + k * C, C)], ii2.at[p], isem).wait()

        def in_issue(k, p):
            pltpu.make_async_copy(
                xj_hbm.at[ij2.at[p]], rb[p], dsem).start()
            pltpu.make_async_copy(
                g_hbm.at[pl.ds(base + k * C, C)], gb[p], dsem).start()

        def in_wait(k, p):
            pltpu.make_async_copy(
                xj_hbm.at[ij2.at[p]], rb[p], dsem).wait()
            pltpu.make_async_copy(
                g_hbm.at[pl.ds(base + k * C, C)], gb[p], dsem).wait()

        def mul_scatter(p):
            @pl.loop(0, C, step=8)
            def _rows(r):
                for dr in range(8):
                    for t in range(F // 16):
                        s0, s1 = pl.ds(r + dr, 1), pl.ds(16 * t, 16)
                        gb[p].at[s0, s1][...] = (
                            gb[p].at[s0, s1][...] * rb[p].at[s0, s1][...])

            pltpu.sync_copy(gb[p], agg_sh.at[ii2.at[p]], add=True)

        def chunk_step(k, p, issue_next):
            if issue_next:
                idx_issue(k + 1, 1 - p)
            in_wait(k, p)
            if issue_next:
                idx_wait(k + 1, 1 - p)
                in_issue(k + 1, 1 - p)
            mul_scatter(p)

        idx_issue(0, 0)
        idx_wait(0, 0)
        in_issue(0, 0)

        @pl.loop(0, NCH - 1, step=2)
        def _pair(k):
            chunk_step(k, 0, True)
            chunk_step(k + 1, 1, True)

        chunk_step(NCH - 1, 0, False)

        plsc.subcore_barrier()

        @pl.when(sid < NS - 1)
        def _f_body():
            pltpu.sync_copy(agg_sh.at[pl.ds(sid * RPS, RPS)],
                            out_hbm.at[pl.ds(cid * N + sid * RPS, RPS)])

        @pl.when(sid == NS - 1)
        def _f_last():
            pltpu.sync_copy(agg_sh.at[pl.ds((NS - 1) * RPS, RPS_LAST)],
                            out_hbm.at[pl.ds(cid * N + (NS - 1) * RPS, RPS_LAST)])

    return ek(g, xj, idx_i, idx_j, zeros_nf)


def kernel(features, descriptors, idx_i, idx_j, params):
    N, F = features.shape
    idx_i = idx_i.astype(jnp.int32)
    idx_j = idx_j.astype(jnp.int32)
    zeros_nf = jnp.zeros((N, F), jnp.float32)

    (g_first,) = _g_all(descriptors, [params[0]['Wg']])
    g_rest = _g_all(descriptors, [p['Wg'] for p in params[1:]])
    g_list = [g_first] + list(g_rest)

    x = features
    outs = []
    xi, xj = _node_pre(x, params[0]['Wi'], params[0]['bi'],
                       params[0]['Wj'], params[0]['bj'])
    for b, p in enumerate(params):
        agg2 = _edge_sc(g_list[b], xj, idx_i, idx_j, zeros_nf)
        if b + 1 < len(params):
            x, xi, xj = _node_post(x, xi, agg2, p, params[b + 1])
        else:
            x = _node_post(x, xi, agg2, p)
        outs.append(x)
    return tuple(outs)
